# Initial kernel scaffold; baseline (speedup 1.0000x reference)
import jax
import jax.numpy as jnp
from jax.experimental import pallas as pl


def kernel(t, gamma):
    raise NotImplementedError("write your pallas kernel here")


# speedup vs baseline: 4.5304x; 4.5304x over previous
import functools
import jax
import jax.numpy as jnp
from jax import lax
from jax.experimental import pallas as pl
from jax.experimental.pallas import tpu as pltpu
from jax.experimental.pallas import tpu_sc as plsc

_TIMESTEPS = 1000
_MAGIC = 8388608.0
_B = 16384
_LANES = 16

_info = plsc.get_sparse_core_info()
_NC, _NS = _info.num_cores, _info.num_subcores
_NW = _NC * _NS
_CHUNK = _B // _NW
_GAMMA_PAD = 1024


def _body(t_hbm, gamma_hbm, out_hbm, gamma_v, t_v, out_v):
    wid = lax.axis_index("s") * _NC + lax.axis_index("c")
    base = wid * _CHUNK
    pltpu.sync_copy(gamma_hbm, gamma_v)
    pltpu.sync_copy(t_hbm.at[pl.ds(base, _CHUNK)], t_v)
    for i in range(_CHUNK // _LANES):
        tv = t_v[pl.ds(i * _LANES, _LANES)]
        xf = (tv * float(_TIMESTEPS) + _MAGIC) - _MAGIC
        idx = xf.astype(jnp.int32)
        out_v[pl.ds(i * _LANES, _LANES)] = plsc.load_gather(gamma_v, [idx])
    pltpu.sync_copy(out_v, out_hbm.at[pl.ds(base, _CHUNK)])


_mesh = plsc.VectorSubcoreMesh(core_axis_name="c", subcore_axis_name="s")

_sc_lookup = pl.kernel(
    _body,
    out_type=jax.ShapeDtypeStruct((_B,), jnp.float32),
    mesh=_mesh,
    scratch_types=[
        pltpu.VMEM((_GAMMA_PAD,), jnp.float32),
        pltpu.VMEM((_CHUNK,), jnp.float32),
        pltpu.VMEM((_CHUNK,), jnp.float32),
    ],
    compiler_params=pltpu.CompilerParams(needs_layout_passes=False),
)


@jax.jit
def kernel(t, gamma):
    gamma_p = jnp.zeros((_GAMMA_PAD,), jnp.float32).at[: gamma.shape[0]].set(gamma)
    out = _sc_lookup(t.reshape(_B), gamma_p)
    return out.reshape(t.shape)


# speedup vs baseline: 4.6275x; 1.0214x over previous
import functools
import jax
import jax.numpy as jnp
from jax import lax
from jax.experimental import pallas as pl
from jax.experimental.pallas import tpu as pltpu
from jax.experimental.pallas import tpu_sc as plsc

_TIMESTEPS = 1000
_MAGIC = 8388608.0
_B = 16384
_LANES = 16

_info = plsc.get_sparse_core_info()
_NC, _NS = _info.num_cores, _info.num_subcores
_NW = _NC * _NS
_CHUNK = _B // _NW
_GAMMA_N = 1001


def _body(t_hbm, gamma_hbm, out_hbm, gamma_v, t_v, out_v, sem_g, sem_t):
    wid = lax.axis_index("s") * _NC + lax.axis_index("c")
    base = wid * _CHUNK
    cp_g = pltpu.async_copy(gamma_hbm, gamma_v, sem_g)
    cp_t = pltpu.async_copy(t_hbm.at[pl.ds(base, _CHUNK)], t_v, sem_t)
    cp_g.wait()
    cp_t.wait()
    for i in range(_CHUNK // _LANES):
        tv = t_v[pl.ds(i * _LANES, _LANES)]
        xf = (tv * float(_TIMESTEPS) + _MAGIC) - _MAGIC
        idx = xf.astype(jnp.int32)
        out_v[pl.ds(i * _LANES, _LANES)] = plsc.load_gather(gamma_v, [idx])
    pltpu.sync_copy(out_v, out_hbm.at[pl.ds(base, _CHUNK)])


_mesh = plsc.VectorSubcoreMesh(core_axis_name="c", subcore_axis_name="s")

_sc_lookup = pl.kernel(
    _body,
    out_type=jax.ShapeDtypeStruct((_B,), jnp.float32),
    mesh=_mesh,
    scratch_types=[
        pltpu.VMEM((_GAMMA_N,), jnp.float32),
        pltpu.VMEM((_CHUNK,), jnp.float32),
        pltpu.VMEM((_CHUNK,), jnp.float32),
        pltpu.SemaphoreType.DMA,
        pltpu.SemaphoreType.DMA,
    ],
    compiler_params=pltpu.CompilerParams(needs_layout_passes=False),
)


@jax.jit
def kernel(t, gamma):
    out = _sc_lookup(t.reshape(_B), gamma)
    return out.reshape(t.shape)


# speedup vs baseline: 4.6797x; 1.0113x over previous
import functools
import jax
import jax.numpy as jnp
from jax import lax
from jax.experimental import pallas as pl
from jax.experimental.pallas import tpu as pltpu
from jax.experimental.pallas import tpu_sc as plsc

_TIMESTEPS = 1000
_MAGIC = 8388608.0
_B = 16384
_LANES = 16

_info = plsc.get_sparse_core_info()
_NC, _NS = _info.num_cores, _info.num_subcores
_NW = _NC * _NS
_CHUNK = _B // _NW
_GAMMA_N = 1001


def _body(t_hbm, gamma_hbm, out_hbm, gamma_v, t_v, out_v, sem_g, sem_t):
    wid = lax.axis_index("s") * _NC + lax.axis_index("c")
    base = wid * _CHUNK
    cp_g = pltpu.async_copy(gamma_hbm, gamma_v, sem_g)
    cp_t = pltpu.async_copy(t_hbm.at[pl.ds(base, _CHUNK)], t_v, sem_t)
    cp_g.wait()
    cp_t.wait()
    half = _CHUNK // 2
    for i in range(half // _LANES):
        tv = t_v[pl.ds(i * _LANES, _LANES)]
        xf = (tv * float(_TIMESTEPS) + _MAGIC) - _MAGIC
        idx = xf.astype(jnp.int32)
        out_v[pl.ds(i * _LANES, _LANES)] = plsc.load_gather(gamma_v, [idx])
    cp_o1 = pltpu.async_copy(
        out_v.at[pl.ds(0, half)], out_hbm.at[pl.ds(base, half)], sem_g
    )
    for i in range(half // _LANES, _CHUNK // _LANES):
        tv = t_v[pl.ds(i * _LANES, _LANES)]
        xf = (tv * float(_TIMESTEPS) + _MAGIC) - _MAGIC
        idx = xf.astype(jnp.int32)
        out_v[pl.ds(i * _LANES, _LANES)] = plsc.load_gather(gamma_v, [idx])
    cp_o2 = pltpu.async_copy(
        out_v.at[pl.ds(half, half)], out_hbm.at[pl.ds(base + half, half)], sem_t
    )
    cp_o1.wait()
    cp_o2.wait()


_mesh = plsc.VectorSubcoreMesh(core_axis_name="c", subcore_axis_name="s")

_sc_lookup = pl.kernel(
    _body,
    out_type=jax.ShapeDtypeStruct((_B,), jnp.float32),
    mesh=_mesh,
    scratch_types=[
        pltpu.VMEM((_GAMMA_N,), jnp.float32),
        pltpu.VMEM((_CHUNK,), jnp.float32),
        pltpu.VMEM((_CHUNK,), jnp.float32),
        pltpu.SemaphoreType.DMA,
        pltpu.SemaphoreType.DMA,
    ],
    compiler_params=pltpu.CompilerParams(needs_layout_passes=False),
)


@jax.jit
def kernel(t, gamma):
    out = _sc_lookup(t.reshape(_B), gamma)
    return out.reshape(t.shape)


# speedup vs baseline: 4.7946x; 1.0246x over previous
import functools
import jax
import jax.numpy as jnp
from jax import lax
from jax.experimental import pallas as pl
from jax.experimental.pallas import tpu as pltpu
from jax.experimental.pallas import tpu_sc as plsc

_TIMESTEPS = 1000
_MAGIC = 8388608.0
_B = 16384
_LANES = 16

_info = plsc.get_sparse_core_info()
_NC, _NS = _info.num_cores, _info.num_subcores
_NW = _NC * _NS
_CHUNK = _B // _NW
_GAMMA_N = 1001


def _body(t_hbm, gamma_hbm, out_hbm, gamma_v, t_v, out_v, sem_g, sem_t):
    wid = lax.axis_index("s") * _NC + lax.axis_index("c")
    base = wid * _CHUNK
    cp_g = pltpu.async_copy(gamma_hbm, gamma_v, sem_g)
    cp_t = pltpu.async_copy(t_hbm.at[pl.ds(base, _CHUNK)], t_v, sem_t)
    cp_g.wait()
    cp_t.wait()

    def step(i, carry):
        off = i * _LANES
        tv = t_v[pl.ds(off, _LANES)]
        xf = (tv * float(_TIMESTEPS) + _MAGIC) - _MAGIC
        idx = xf.astype(jnp.int32)
        out_v[pl.ds(off, _LANES)] = plsc.load_gather(gamma_v, [idx])
        return carry

    lax.fori_loop(0, _CHUNK // _LANES, step, 0)
    pltpu.sync_copy(out_v, out_hbm.at[pl.ds(base, _CHUNK)])


_mesh = plsc.VectorSubcoreMesh(core_axis_name="c", subcore_axis_name="s")

_sc_lookup = pl.kernel(
    _body,
    out_type=jax.ShapeDtypeStruct((_B,), jnp.float32),
    mesh=_mesh,
    scratch_types=[
        pltpu.VMEM((_GAMMA_N,), jnp.float32),
        pltpu.VMEM((_CHUNK,), jnp.float32),
        pltpu.VMEM((_CHUNK,), jnp.float32),
        pltpu.SemaphoreType.DMA,
        pltpu.SemaphoreType.DMA,
    ],
    compiler_params=pltpu.CompilerParams(needs_layout_passes=False),
)


@jax.jit
def kernel(t, gamma):
    out = _sc_lookup(t.reshape(_B), gamma)
    return out.reshape(t.shape)
